# baseline (speedup 1.0000x reference)
import functools

import jax
import jax.numpy as jnp
from jax import lax
from jax.experimental import pallas as pl
from jax.experimental.pallas import tpu as pltpu
from jax.experimental.pallas import tpu_sc as plsc

H = 768
L = 16
HV = H // L
NC, NS = 2, 16
NW = NC * NS
K = 16
MAX_POS = 4096
SQRT_H = float(H) ** 0.5
EPS = 1e-12


def _combine_tables(ptab, ttab):
    bp = 512
    npos = ptab.shape[0]

    def body(p_ref, t_ref, o_ref):
        p = p_ref[...]
        o_ref[0] = p + t_ref[0]
        o_ref[1] = p + t_ref[1]

    return pl.pallas_call(
        body,
        grid=(npos // bp,),
        in_specs=[
            pl.BlockSpec((bp, H), lambda i: (i, 0)),
            pl.BlockSpec((2, H), lambda i: (0, 0)),
        ],
        out_specs=pl.BlockSpec((2, bp, H), lambda i: (0, i, 0)),
        out_shape=jax.ShapeDtypeStruct((2, npos, H), jnp.float32),
    )(ptab, ttab)


def _lane_sum(v):
    for sh in (8, 4, 2, 1):
        idx = lax.iota(jnp.int32, L) ^ sh
        v = v + v.at[idx].get(mode="promise_in_bounds")
    return v


def _embed_ln_sc(word_ids, pos_ids, tt_ids, wtab, ctab, scale, bias):
    tok = word_ids.shape[0]
    tpw = tok // NW
    nchunk = tpw // K

    mesh = plsc.VectorSubcoreMesh(core_axis_name="c", subcore_axis_name="s")

    @functools.partial(
        pl.kernel,
        mesh=mesh,
        out_type=jax.ShapeDtypeStruct((tok, H), jnp.float32),
        scratch_types=[
            pltpu.VMEM((tpw,), jnp.int32),
            pltpu.VMEM((tpw,), jnp.int32),
            pltpu.VMEM((tpw,), jnp.int32),
            pltpu.VMEM((tpw,), jnp.int32),
            pltpu.VMEM((K, H), jnp.float32),
            pltpu.VMEM((K, H), jnp.float32),
            pltpu.VMEM((K, H), jnp.float32),
            pltpu.VMEM((K, H), jnp.float32),
            pltpu.VMEM((K, H), jnp.float32),
            pltpu.VMEM((K, H), jnp.float32),
            pltpu.VMEM((H,), jnp.float32),
            pltpu.VMEM((H,), jnp.float32),
            pltpu.VMEM((K, L), jnp.float32),
            pltpu.VMEM((K, L), jnp.float32),
            pltpu.SemaphoreType.DMA,
            pltpu.SemaphoreType.DMA,
            pltpu.SemaphoreType.DMA,
            pltpu.SemaphoreType.DMA,
            pltpu.SemaphoreType.DMA,
            pltpu.SemaphoreType.DMA,
        ],
    )
    def body(wid_hbm, pid_hbm, tid_hbm, wtab_hbm, ctab_hbm,
             sc_hbm, bi_hbm, out_hbm,
             wid_v, pid_v, tid_v, cid_v, Wb0, Wb1, Cb0, Cb1, Ob0, Ob1,
             sc_v, bi_v, ybuf, mbuf, gw0, gw1, gc0, gc1, wb0, wb1):
        w = lax.axis_index("s") * NC + lax.axis_index("c")
        base = w * tpw
        pltpu.sync_copy(wid_hbm.at[pl.ds(base, tpw)], wid_v)
        pltpu.sync_copy(pid_hbm.at[pl.ds(base, tpw)], pid_v)
        pltpu.sync_copy(tid_hbm.at[pl.ds(base, tpw)], tid_v)
        pltpu.sync_copy(sc_hbm, sc_v)
        pltpu.sync_copy(bi_hbm, bi_v)

        @plsc.parallel_loop(0, tpw // L, 1)
        def _(j):
            sl = pl.ds(j * L, L)
            cid_v[sl] = tid_v[sl] * MAX_POS + pid_v[sl]

        Wb = (Wb0, Wb1)
        Cb = (Cb0, Cb1)
        Ob = (Ob0, Ob1)
        gw = (gw0, gw1)
        gc = (gc0, gc1)
        wb = (wb0, wb1)

        def gathers(c, s):
            off = c * K
            pltpu.make_async_copy(
                wtab_hbm.at[wid_v.at[pl.ds(off, K)]], Wb[s], gw[s]).start()
            pltpu.make_async_copy(
                ctab_hbm.at[cid_v.at[pl.ds(off, K)]], Cb[s], gc[s]).start()

        def wait_gathers(c, s):
            off = c * K
            pltpu.make_async_copy(
                wtab_hbm.at[wid_v.at[pl.ds(off, K)]], Wb[s], gw[s]).wait()
            pltpu.make_async_copy(
                ctab_hbm.at[cid_v.at[pl.ds(off, K)]], Cb[s], gc[s]).wait()

        def compute(c, s):
            Wc, Cc, Oc = Wb[s], Cb[s], Ob[s]

            @plsc.parallel_loop(0, K, 1, unroll=3)
            def token(t):
                acc_a = jnp.zeros((L,), jnp.float32)
                acc_b = jnp.zeros((L,), jnp.float32)
                acc2_a = jnp.zeros((L,), jnp.float32)
                acc2_b = jnp.zeros((L,), jnp.float32)
                for j in range(HV):
                    wv = Wc[t, pl.ds(j * L, L)]
                    cv = Cc[t, pl.ds(j * L, L)]
                    h = wv * SQRT_H + cv
                    Oc[t, pl.ds(j * L, L)] = h
                    if j % 2 == 0:
                        acc_a = acc_a + h
                        acc2_a = acc2_a + h * h
                    else:
                        acc_b = acc_b + h
                        acc2_b = acc2_b + h * h
                s1 = _lane_sum(acc_a + acc_b)[0]
                s2 = _lane_sum(acc2_a + acc2_b)[0]
                mean = s1 * (1.0 / H)
                var = s2 * (1.0 / H) - mean * mean
                x = var + EPS
                i = lax.bitcast_convert_type(x, jnp.int32)
                i = 0x5F3759DF - lax.shift_right_logical(i, 1)
                ys = lax.bitcast_convert_type(i, jnp.float32)
                hx = x * 0.5
                for _ in range(3):
                    ys = ys * (1.5 - hx * ys * ys)
                ybuf[t, pl.ds(0, L)] = jnp.full((L,), ys, jnp.float32)
                mbuf[t, pl.ds(0, L)] = jnp.full((L,), mean, jnp.float32)

            ng = 6
            jg = HV // ng
            for g in range(ng):
                scg = [sc_v[pl.ds((g * jg + j) * L, L)] for j in range(jg)]
                big = [bi_v[pl.ds((g * jg + j) * L, L)] for j in range(jg)]

                @plsc.parallel_loop(0, K, 1, unroll=4)
                def norm(t):
                    yv = ybuf[t, pl.ds(0, L)]
                    mv = mbuf[t, pl.ds(0, L)]
                    for j in range(jg):
                        col = g * jg + j
                        h = Oc[t, pl.ds(col * L, L)]
                        a = yv * scg[j]
                        b = big[j] - mv * a
                        Oc[t, pl.ds(col * L, L)] = h * a + b

        def writeback(c, s):
            pltpu.make_async_copy(
                Ob[s], out_hbm.at[pl.ds(base + c * K, K)], wb[s]).start()

        def wait_writeback(c, s):
            pltpu.make_async_copy(
                Ob[s], out_hbm.at[pl.ds(base + c * K, K)], wb[s]).wait()

        gathers(0, 0)
        gathers(1, 1)

        def pair(i, carry):
            for s in (0, 1):
                c = 2 * i + s
                wait_gathers(c, s)

                @pl.when(c >= 2)
                def _():
                    wait_writeback(c - 2, s)

                compute(c, s)
                writeback(c, s)

                @pl.when(c + 2 < nchunk)
                def _():
                    gathers(c + 2, s)

            return carry

        lax.fori_loop(0, nchunk // 2, pair, 0)
        wait_writeback(nchunk - 2, 0)
        wait_writeback(nchunk - 1, 1)

    return body(word_ids, pos_ids, tt_ids, wtab, ctab, scale, bias)


def kernel(input_ids, token_type_ids, position_ids, attention_mask,
           word_embeddings, position_embeddings, token_type_embeddings,
           ln_scale, ln_bias):
    b, s = input_ids.shape
    wids = input_ids.reshape(-1).astype(jnp.int32)
    pids = position_ids.reshape(-1).astype(jnp.int32)
    tids = token_type_ids.reshape(-1).astype(jnp.int32)
    ctab = _combine_tables(position_embeddings, token_type_embeddings)
    out = _embed_ln_sc(wids, pids, tids, word_embeddings,
                       ctab.reshape(-1, H), ln_scale, ln_bias)
    return out.reshape(b, s, H)


# speedup vs baseline: 1.3445x; 1.3445x over previous
import functools

import jax
import jax.numpy as jnp
from jax import lax
from jax.experimental import pallas as pl
from jax.experimental.pallas import tpu as pltpu
from jax.experimental.pallas import tpu_sc as plsc

H = 768
L = 16
HV = H // L
NC, NS = 2, 16
NW = NC * NS
K = 16
MAX_POS = 4096
SQRT_H = float(H) ** 0.5
EPS = 1e-12


def _combine_tables(ptab, ttab):
    bp = 512
    npos = ptab.shape[0]

    def body(p_ref, t_ref, o_ref):
        p = p_ref[...]
        o_ref[0] = p + t_ref[0]
        o_ref[1] = p + t_ref[1]

    return pl.pallas_call(
        body,
        grid=(npos // bp,),
        in_specs=[
            pl.BlockSpec((bp, H), lambda i: (i, 0)),
            pl.BlockSpec((2, H), lambda i: (0, 0)),
        ],
        out_specs=pl.BlockSpec((2, bp, H), lambda i: (0, i, 0)),
        out_shape=jax.ShapeDtypeStruct((2, npos, H), jnp.float32),
    )(ptab, ttab)


def _lane_sum(v):
    for sh in (8, 4, 2, 1):
        idx = lax.iota(jnp.int32, L) ^ sh
        v = v + v.at[idx].get(mode="promise_in_bounds")
    return v


def _embed_ln_sc(word_ids, pos_ids, tt_ids, wtab, ctab, scale, bias):
    tok = word_ids.shape[0]
    tpw = tok // NW
    nchunk = tpw // K

    mesh = plsc.VectorSubcoreMesh(core_axis_name="c", subcore_axis_name="s")

    @functools.partial(
        pl.kernel,
        mesh=mesh,
        out_type=jax.ShapeDtypeStruct((tok, H), jnp.float32),
        scratch_types=[
            pltpu.VMEM((tpw,), jnp.int32),
            pltpu.VMEM((tpw,), jnp.int32),
            pltpu.VMEM((tpw,), jnp.int32),
            pltpu.VMEM((tpw,), jnp.int32),
            pltpu.VMEM((K, H), jnp.float32),
            pltpu.VMEM((K, H), jnp.float32),
            pltpu.VMEM((K, H), jnp.float32),
            pltpu.VMEM((K, H), jnp.float32),
            pltpu.VMEM((K, H), jnp.float32),
            pltpu.VMEM((K, H), jnp.float32),
            pltpu.VMEM((H,), jnp.float32),
            pltpu.VMEM((H,), jnp.float32),
            pltpu.VMEM((K, L), jnp.float32),
            pltpu.VMEM((K, L), jnp.float32),
            pltpu.SemaphoreType.DMA,
            pltpu.SemaphoreType.DMA,
            pltpu.SemaphoreType.DMA,
            pltpu.SemaphoreType.DMA,
            pltpu.SemaphoreType.DMA,
            pltpu.SemaphoreType.DMA,
        ],
    )
    def body(wid_hbm, pid_hbm, tid_hbm, wtab_hbm, ctab_hbm,
             sc_hbm, bi_hbm, out_hbm,
             wid_v, pid_v, tid_v, cid_v, Wb0, Wb1, Cb0, Cb1, Ob0, Ob1,
             sc_v, bi_v, ybuf, mbuf, gw0, gw1, gc0, gc1, wb0, wb1):
        w = lax.axis_index("s") * NC + lax.axis_index("c")
        base = w * tpw
        pltpu.sync_copy(wid_hbm.at[pl.ds(base, tpw)], wid_v)
        pltpu.sync_copy(pid_hbm.at[pl.ds(base, tpw)], pid_v)
        pltpu.sync_copy(tid_hbm.at[pl.ds(base, tpw)], tid_v)
        pltpu.sync_copy(sc_hbm, sc_v)
        pltpu.sync_copy(bi_hbm, bi_v)

        @plsc.parallel_loop(0, tpw // L, 1)
        def _(j):
            sl = pl.ds(j * L, L)
            cid_v[sl] = tid_v[sl] * MAX_POS + pid_v[sl]

        Wb = (Wb0, Wb1)
        Cb = (Cb0, Cb1)
        Ob = (Ob0, Ob1)
        gw = (gw0, gw1)
        gc = (gc0, gc1)
        wb = (wb0, wb1)

        def gathers(c, s):
            off = c * K
            pltpu.make_async_copy(
                wtab_hbm.at[wid_v.at[pl.ds(off, K)]], Wb[s], gw[s]).start()
            pltpu.make_async_copy(
                ctab_hbm.at[cid_v.at[pl.ds(off, K)]], Cb[s], gc[s]).start()

        def wait_gathers(c, s):
            off = c * K
            pltpu.make_async_copy(
                wtab_hbm.at[wid_v.at[pl.ds(off, K)]], Wb[s], gw[s]).wait()
            pltpu.make_async_copy(
                ctab_hbm.at[cid_v.at[pl.ds(off, K)]], Cb[s], gc[s]).wait()

        def compute(c, s):
            Wc, Cc, Oc = Wb[s], Cb[s], Ob[s]

            @plsc.parallel_loop(0, K, 1, unroll=4)
            def token(t):
                acc_a = jnp.zeros((L,), jnp.float32)
                acc_b = jnp.zeros((L,), jnp.float32)
                acc2_a = jnp.zeros((L,), jnp.float32)
                acc2_b = jnp.zeros((L,), jnp.float32)
                for j in range(HV):
                    wv = Wc[t, pl.ds(j * L, L)]
                    cv = Cc[t, pl.ds(j * L, L)]
                    h = wv * SQRT_H + cv
                    Oc[t, pl.ds(j * L, L)] = h
                    if j % 2 == 0:
                        acc_a = acc_a + h
                        acc2_a = acc2_a + h * h
                    else:
                        acc_b = acc_b + h
                        acc2_b = acc2_b + h * h
                s1 = _lane_sum(acc_a + acc_b)[0]
                s2 = _lane_sum(acc2_a + acc2_b)[0]
                mean = s1 * (1.0 / H)
                var = s2 * (1.0 / H) - mean * mean
                x = var + EPS
                i = lax.bitcast_convert_type(x, jnp.int32)
                i = 0x5F3759DF - lax.shift_right_logical(i, 1)
                ys = lax.bitcast_convert_type(i, jnp.float32)
                hx = x * 0.5
                for _ in range(3):
                    ys = ys * (1.5 - hx * ys * ys)
                ybuf[t, pl.ds(0, L)] = jnp.full((L,), ys, jnp.float32)
                mbuf[t, pl.ds(0, L)] = jnp.full((L,), mean, jnp.float32)

            ng = 6
            jg = HV // ng
            for g in range(ng):
                scg = [sc_v[pl.ds((g * jg + j) * L, L)] for j in range(jg)]
                big = [bi_v[pl.ds((g * jg + j) * L, L)] for j in range(jg)]

                @plsc.parallel_loop(0, K, 1, unroll=2)
                def norm(t):
                    yv = ybuf[t, pl.ds(0, L)]
                    mv = mbuf[t, pl.ds(0, L)]
                    for j in range(jg):
                        col = g * jg + j
                        h = Oc[t, pl.ds(col * L, L)]
                        a = yv * scg[j]
                        b = big[j] - mv * a
                        Oc[t, pl.ds(col * L, L)] = h * a + b

        def writeback(c, s):
            pltpu.make_async_copy(
                Ob[s], out_hbm.at[pl.ds(base + c * K, K)], wb[s]).start()

        def wait_writeback(c, s):
            pltpu.make_async_copy(
                Ob[s], out_hbm.at[pl.ds(base + c * K, K)], wb[s]).wait()

        gathers(0, 0)
        gathers(1, 1)

        def pair(i, carry):
            for s in (0, 1):
                c = 2 * i + s
                wait_gathers(c, s)

                @pl.when(c >= 2)
                def _():
                    wait_writeback(c - 2, s)

                compute(c, s)
                writeback(c, s)

                @pl.when(c + 2 < nchunk)
                def _():
                    gathers(c + 2, s)

            return carry

        lax.fori_loop(0, nchunk // 2, pair, 0)
        wait_writeback(nchunk - 2, 0)
        wait_writeback(nchunk - 1, 1)

    return body(word_ids, pos_ids, tt_ids, wtab, ctab, scale, bias)


def kernel(input_ids, token_type_ids, position_ids, attention_mask,
           word_embeddings, position_embeddings, token_type_embeddings,
           ln_scale, ln_bias):
    b, s = input_ids.shape
    wids = input_ids.reshape(-1).astype(jnp.int32)
    pids = position_ids.reshape(-1).astype(jnp.int32)
    tids = token_type_ids.reshape(-1).astype(jnp.int32)
    ctab = _combine_tables(position_embeddings, token_type_embeddings)
    out = _embed_ln_sc(wids, pids, tids, word_embeddings,
                       ctab.reshape(-1, H), ln_scale, ln_bias)
    return out.reshape(b, s, H)
